# Initial kernel scaffold; baseline (speedup 1.0000x reference)
import jax
import jax.numpy as jnp
import numpy as np
from jax.experimental import pallas as pl

_N = 10000
_B = 64

_DEG_HIST = np.array([0]*28 + [200,600,1200,1800,2400,1800,1200,600,200] + [0]*28, dtype=np.float64)
_bins = np.arange(_DEG_HIST.shape[0], dtype=np.float64)
_AVG_DEG_LOG = float((np.log(_bins + 1.0) * _DEG_HIST).sum() / _DEG_HIST.sum())


def _mlp_kernel(p_ref, w1_ref, b1_ref, w2_ref, b2_ref, o_ref):
    h = jnp.maximum(p_ref[...] @ w1_ref[...] + b1_ref[...], 0.0)
    o_ref[...] = h @ w2_ref[...] + b2_ref[...]


def _pna_conv(x, src, dst, W_pre, b_pre, W_post, b_post, W_lin, b_lin):
    h = jnp.concatenate([jnp.take(x, dst, axis=0), jnp.take(x, src, axis=0)], axis=-1)
    m = h @ W_pre + b_pre
    count = jax.ops.segment_sum(jnp.ones((m.shape[0],), dtype=m.dtype), dst, num_segments=_N)
    cnt = jnp.maximum(count, 1.0)[:, None]
    mean = jax.ops.segment_sum(m, dst, num_segments=_N) / cnt
    mn = jax.ops.segment_min(m, dst, num_segments=_N)
    mx = jax.ops.segment_max(m, dst, num_segments=_N)
    has = (count > 0)[:, None]
    mn = jnp.where(has, mn, 0.0)
    mx = jnp.where(has, mx, 0.0)
    mean2 = jax.ops.segment_sum(m * m, dst, num_segments=_N) / cnt
    std = jnp.sqrt(jax.nn.relu(mean2 - mean * mean) + 1e-5)
    agg = jnp.concatenate([mean, mn, mx, std], axis=-1)
    d = jnp.maximum(count, 1.0)
    amp = (jnp.log(d + 1.0) / _AVG_DEG_LOG)[:, None]
    att = (_AVG_DEG_LOG / jnp.log(d + 1.0))[:, None]
    scaled = jnp.concatenate([agg, agg * amp, agg * att], axis=-1)
    out = jnp.concatenate([x, scaled], axis=-1) @ W_post + b_post
    return out @ W_lin + b_lin


def kernel(x, edge_index, batch, W_pre_0, b_pre_0, W_post_0, b_post_0, W_lin_0, b_lin_0, W_pre_1, b_pre_1, W_post_1, b_post_1, W_lin_1, b_lin_1, W_mol1, b_mol1, W_mol2, b_mol2):
    src = edge_index[0]
    dst = edge_index[1]
    h = _pna_conv(x, src, dst, W_pre_0, b_pre_0, W_post_0, b_post_0, W_lin_0, b_lin_0)
    h = jax.nn.relu(h)
    h = _pna_conv(h, src, dst, W_pre_1, b_pre_1, W_post_1, b_post_1, W_lin_1, b_lin_1)
    gcnt = jnp.maximum(jax.ops.segment_sum(jnp.ones((_N,), dtype=h.dtype), batch, num_segments=_B), 1.0)[:, None]
    pooled = jnp.segment_sum if False else jax.ops.segment_sum(h, batch, num_segments=_B) / gcnt
    out = pl.pallas_call(
        _mlp_kernel,
        out_shape=jax.ShapeDtypeStruct((_B, W_mol2.shape[1]), jnp.float32),
    )(pooled, W_mol1, b_mol1, W_mol2, b_mol2)
    return out


# speedup vs baseline: 1.0008x; 1.0008x over previous
import jax
import jax.numpy as jnp
import numpy as np
from jax.experimental import pallas as pl

_N = 10000
_B = 64

_DEG_HIST = np.array([0]*28 + [200,600,1200,1800,2400,1800,1200,600,200] + [0]*28, dtype=np.float64)
_bins = np.arange(_DEG_HIST.shape[0], dtype=np.float64)
_AVG_DEG_LOG = float((np.log(_bins + 1.0) * _DEG_HIST).sum() / _DEG_HIST.sum())


def _mlp_kernel(p_ref, w1_ref, b1_ref, w2_ref, b2_ref, o_ref):
    h = jnp.maximum(p_ref[...] @ w1_ref[...] + b1_ref[...], 0.0)
    o_ref[...] = h @ w2_ref[...] + b2_ref[...]


def _pna_conv(x, src, dst, W_pre, b_pre, W_post, b_post, W_lin, b_lin):
    h = jnp.concatenate([jnp.take(x, dst, axis=0), jnp.take(x, src, axis=0)], axis=-1)
    m = h @ W_pre + b_pre
    count = jax.ops.segment_sum(jnp.ones((m.shape[0],), dtype=m.dtype), dst, num_segments=_N)
    cnt = jnp.maximum(count, 1.0)[:, None]
    mean = jax.ops.segment_sum(m, dst, num_segments=_N) / cnt
    mn = jax.ops.segment_min(m, dst, num_segments=_N)
    mx = jax.ops.segment_max(m, dst, num_segments=_N)
    has = (count > 0)[:, None]
    mn = jnp.where(has, mn, 0.0)
    mx = jnp.where(has, mx, 0.0)
    mean2 = jax.ops.segment_sum(m * m, dst, num_segments=_N) / cnt
    std = jnp.sqrt(jax.nn.relu(mean2 - mean * mean) + 1e-5)
    agg = jnp.concatenate([mean, mn, mx, std], axis=-1)
    d = jnp.maximum(count, 1.0)
    amp = (jnp.log(d + 1.0) / _AVG_DEG_LOG)[:, None]
    att = (_AVG_DEG_LOG / jnp.log(d + 1.0))[:, None]
    scaled = jnp.concatenate([agg, agg * amp, agg * att], axis=-1)
    out = jnp.concatenate([x, scaled], axis=-1) @ W_post + b_post
    return out @ W_lin + b_lin


def kernel(x, edge_index, batch, W_pre_0, b_pre_0, W_post_0, b_post_0, W_lin_0, b_lin_0, W_pre_1, b_pre_1, W_post_1, b_post_1, W_lin_1, b_lin_1, W_mol1, b_mol1, W_mol2, b_mol2):
    src = edge_index[0]
    dst = edge_index[1]
    h = _pna_conv(x, src, dst, W_pre_0, b_pre_0, W_post_0, b_post_0, W_lin_0, b_lin_0)
    h = jax.nn.relu(h)
    h = _pna_conv(h, src, dst, W_pre_1, b_pre_1, W_post_1, b_post_1, W_lin_1, b_lin_1)
    gcnt = jnp.maximum(jax.ops.segment_sum(jnp.ones((_N,), dtype=h.dtype), batch, num_segments=_B), 1.0)[:, None]
    pooled = jax.ops.segment_sum(h, batch, num_segments=_B) / gcnt
    out = pl.pallas_call(
        _mlp_kernel,
        out_shape=jax.ShapeDtypeStruct((_B, W_mol2.shape[1]), jnp.float32),
    )(pooled, W_mol1, b_mol1, W_mol2, b_mol2)
    return out


# speedup vs baseline: 4.6403x; 4.6367x over previous
import functools

import jax
import jax.numpy as jnp
import numpy as np
from jax import lax
from jax.experimental import pallas as pl
from jax.experimental.pallas import tpu as pltpu
from jax.experimental.pallas import tpu_sc as plsc

_N = 10000
_E = 320000
_F = 128
_B = 64

_NCH = 64
_CROWS = 160
_NPAD = _NCH * _CROWS
_NW = 32
_EP = _E // _NW
_NG = _EP // 16
_CAP = 384
_LCAP = 5888
_RB = 128

_DEG_HIST = np.array([0]*28 + [200,600,1200,1800,2400,1800,1200,600,200] + [0]*28, dtype=np.float64)
_bins = np.arange(_DEG_HIST.shape[0], dtype=np.float64)
_AVG_DEG_LOG = float((np.log(_bins + 1.0) * _DEG_HIST).sum() / _DEG_HIST.sum())

_PREC = jax.lax.Precision.HIGHEST


def _dot(a, b):
    return jnp.dot(a, b, precision=_PREC, preferred_element_type=jnp.float32)


_RBLK = 1024
_NBLK = _NPAD // _RBLK


def _t1_body(x_ref, w1_ref, b1_ref, w2_ref, a_ref, c_ref):
    xb = x_ref[...]
    a_ref[...] = _dot(xb, w1_ref[...]) + b1_ref[...]
    c_ref[...] = _dot(xb, w2_ref[...])


def _t1_call(x_pad, W1, b1r, W2):
    full = lambda shp: pl.BlockSpec(shp, lambda i: (0, 0))
    row = pl.BlockSpec((_RBLK, _F), lambda i: (i, 0))
    return pl.pallas_call(
        _t1_body,
        grid=(_NBLK,),
        in_specs=[row, full((_F, _F)), full((1, _F)), full((_F, _F))],
        out_specs=[row, row],
        out_shape=[jax.ShapeDtypeStruct((_NPAD, _F), jnp.float32)] * 2,
    )(x_pad, W1, b1r, W2)


def _combine_body(layer0, x_ref, a_ref, s1_ref, s2_ref, mn_ref, mx_ref,
                  cnt_ref, wp_ref, bp_ref, wl_ref, bl_ref, *rest):
    if layer0:
        w1n_ref, b1n_ref, w2n_ref, h_ref, an_ref, cn_ref = rest
    else:
        (h_ref,) = rest
    xb = x_ref[...]
    A = a_ref[...]
    S1 = s1_ref[...]
    S2 = s2_ref[...]
    cnt = cnt_ref[...]
    has = cnt > 0.0
    cntc = jnp.maximum(cnt, 1.0)
    inv = 1.0 / cntc
    zero = jnp.zeros_like(A)
    mean = jnp.where(has, A + S1 * inv, zero)
    mn = jnp.where(has, A + mn_ref[...], zero)
    mx = jnp.where(has, A + mx_ref[...], zero)
    mean2 = jnp.where(has, A * A + (2.0 * A * S1 + S2) * inv, zero)
    std = jnp.sqrt(jnp.maximum(mean2 - mean * mean, 0.0) + 1e-5)
    agg = jnp.concatenate([mean, mn, mx, std], axis=-1)
    lg = jnp.log(cntc + 1.0)
    amp = lg * (1.0 / _AVG_DEG_LOG)
    att = _AVG_DEG_LOG / lg
    wp = wp_ref[...]
    out = (_dot(xb, wp[0:_F])
           + _dot(agg, wp[_F:5 * _F])
           + _dot(agg * amp, wp[5 * _F:9 * _F])
           + _dot(agg * att, wp[9 * _F:13 * _F])
           + bp_ref[...])
    out = _dot(out, wl_ref[...]) + bl_ref[...]
    if layer0:
        h = jnp.maximum(out, 0.0)
        h_ref[...] = h
        an_ref[...] = _dot(h, w1n_ref[...]) + b1n_ref[...]
        cn_ref[...] = _dot(h, w2n_ref[...])
    else:
        h_ref[...] = out


def _combine_call(layer0, x_pad, A, S1, S2, Mn, Mx, cnt2d, W_post, b_postr,
                  W_lin, b_linr, extra):
    full = lambda shp: pl.BlockSpec(shp, lambda i: (0, 0))
    row = pl.BlockSpec((_RBLK, _F), lambda i: (i, 0))
    col = pl.BlockSpec((_RBLK, 1), lambda i: (i, 0))
    in_specs = [row, row, row, row, row, row, col,
                full((13 * _F, _F)), full((1, _F)), full((_F, _F)), full((1, _F))]
    args = [x_pad, A, S1, S2, Mn, Mx, cnt2d, W_post, b_postr, W_lin, b_linr]
    if layer0:
        in_specs += [full((_F, _F)), full((1, _F)), full((_F, _F))]
        args += list(extra)
        out_specs = [row, row, row]
        out_shape = [jax.ShapeDtypeStruct((_NPAD, _F), jnp.float32)] * 3
    else:
        out_specs = [row]
        out_shape = [jax.ShapeDtypeStruct((_NPAD, _F), jnp.float32)]
    return pl.pallas_call(
        functools.partial(_combine_body, layer0),
        grid=(_NBLK,),
        in_specs=in_specs,
        out_specs=out_specs,
        out_shape=out_shape,
    )(*args)


def _pool_body(h_ref, b_ref, wm1_ref, bm1_ref, wm2_ref, bm2_ref, o_ref,
               psum, pcnt):
    i = pl.program_id(0)

    @pl.when(i == 0)
    def _():
        psum[...] = jnp.zeros_like(psum)
        pcnt[...] = jnp.zeros_like(pcnt)

    hb = h_ref[...]
    bb = b_ref[...]
    onehot = (bb == lax.broadcasted_iota(jnp.int32, (_RBLK, _B), 1)).astype(jnp.float32)
    psum[...] += lax.dot_general(onehot, hb, (((0,), (0,)), ((), ())),
                                 precision=_PREC, preferred_element_type=jnp.float32)
    pcnt[...] += jnp.broadcast_to(jnp.sum(onehot, axis=0)[:, None], (_B, _F))

    @pl.when(i == _NBLK - 1)
    def _():
        pooled = psum[...] / jnp.maximum(pcnt[...], 1.0)
        hmid = jnp.maximum(_dot(pooled, wm1_ref[...]) + bm1_ref[...], 0.0)
        o_ref[...] = _dot(hmid, wm2_ref[...]) + bm2_ref[...]


def _pool_call(h2, batch_pad, W_mol1, b_mol1r, W_mol2, b_mol2r):
    full = lambda shp: pl.BlockSpec(shp, lambda i: (0, 0))
    row = pl.BlockSpec((_RBLK, _F), lambda i: (i, 0))
    col = pl.BlockSpec((_RBLK, 1), lambda i: (i, 0))
    return pl.pallas_call(
        _pool_body,
        grid=(_NBLK,),
        in_specs=[row, col, full((_F, _F)), full((1, _F)), full((_F, _F)),
                  full((1, _F))],
        out_specs=pl.BlockSpec((_B, _F), lambda i: (0, 0)),
        out_shape=jax.ShapeDtypeStruct((_B, _F), jnp.float32),
        scratch_shapes=[pltpu.VMEM((_B, _F), jnp.float32),
                        pltpu.VMEM((_B, _F), jnp.float32)],
    )(h2, batch_pad, W_mol1, b_mol1r, W_mol2, b_mol2r)


_MESH = plsc.VectorSubcoreMesh(core_axis_name="c", subcore_axis_name="s")


def _worker_id():
    return lax.axis_index("s") * 2 + lax.axis_index("c")


def _sread(ref1d, idx):
    return plsc.load_gather(ref1d, [jnp.broadcast_to(idx, (16,))])[0]


@functools.partial(
    pl.kernel,
    out_type=[jax.ShapeDtypeStruct((_NW, _NCH * _CAP), jnp.int32),
              jax.ShapeDtypeStruct((_NW * _NCH,), jnp.int32)],
    mesh=_MESH,
    compiler_params=pltpu.CompilerParams(needs_layout_passes=False),
    scratch_types=[
        pltpu.VMEM((_EP,), jnp.int32),
        pltpu.VMEM((_EP,), jnp.int32),
        pltpu.VMEM((_NCH * _CAP,), jnp.int32),
        pltpu.VMEM((_NCH,), jnp.int32),
        pltpu.VMEM((16,), jnp.int32),
    ],
)
def _bucket_kernel(src_hbm, dst_hbm, bout, cout, sbuf, dbuf, flat, curs, kscr):
    w = _worker_id()
    base = w * _EP
    pltpu.sync_copy(src_hbm.at[pl.ds(base, _EP)], sbuf)
    pltpu.sync_copy(dst_hbm.at[pl.ds(base, _EP)], dbuf)
    zeros16 = jnp.zeros((16,), jnp.int32)
    ones16 = jnp.ones((16,), jnp.int32)
    iota = lax.iota(jnp.int32, 16)
    for i in range(_NCH // 16):
        curs[pl.ds(i * 16, 16)] = zeros16

    def body(g, carry):
        dvec = dbuf[pl.ds(g * 16, 16)]
        svec = sbuf[pl.ds(g * 16, 16)]
        cid = jnp.bitwise_and(dvec, _NCH - 1)
        dl = jnp.right_shift(dvec, 6)
        packed = svec * 256 + dl
        skey, sval = plsc.sort_key_val(cid, packed)
        kscr[pl.ds(0, 16)] = skey
        prev = plsc.load_gather(kscr, [jnp.maximum(iota - 1, 0)])
        boundary = jnp.logical_or(iota == 0, skey != prev)
        segstart = plsc.cummax(jnp.where(boundary, iota, 0))
        rank = iota - segstart
        cur = plsc.load_gather(curs, [skey])
        pos = jnp.minimum(skey * _CAP + cur + rank, skey * _CAP + (_CAP - 1))
        plsc.store_scatter(flat, [pos], sval)
        plsc.addupdate_scatter(curs, [skey], ones16)
        return carry

    lax.fori_loop(0, _NG, body, 0)
    pltpu.sync_copy(flat, bout.at[w])
    pltpu.sync_copy(curs, cout.at[pl.ds(w * _NCH, _NCH)])


@functools.partial(
    pl.kernel,
    out_type=[jax.ShapeDtypeStruct((_CROWS, _NCH, _F), jnp.float32),
              jax.ShapeDtypeStruct((_CROWS, _NCH, _F), jnp.float32),
              jax.ShapeDtypeStruct((_CROWS, _NCH, _F), jnp.float32),
              jax.ShapeDtypeStruct((_CROWS, _NCH, _F), jnp.float32),
              jax.ShapeDtypeStruct((_NCH, _CROWS), jnp.float32)],
    mesh=_MESH,
    compiler_params=pltpu.CompilerParams(needs_layout_passes=False),
    scratch_types=[
        pltpu.VMEM((_CROWS, _F), jnp.float32),
        pltpu.VMEM((_CROWS, _F), jnp.float32),
        pltpu.VMEM((_CROWS, _F), jnp.float32),
        pltpu.VMEM((_CROWS, _F), jnp.float32),
        pltpu.VMEM((_CROWS,), jnp.int32),
        pltpu.VMEM((_CROWS,), jnp.float32),
        pltpu.VMEM((_NW * _CAP,), jnp.int32),
        pltpu.VMEM((_LCAP,), jnp.int32),
        pltpu.VMEM((_LCAP,), jnp.int32),
        pltpu.VMEM((_RB, _F), jnp.float32),
        pltpu.VMEM((_NW * _NCH,), jnp.int32),
        pltpu.SemaphoreType.DMA,
    ],
)
def _accum_kernel(c_hbm, bkt_hbm, cnt_hbm, s1o, s2o, mno, mxo, cnto,
                  s1, s2, mn, mx, cli, clf, lists, sidx, dlv, rows, cstage,
                  sem):
    w = _worker_id()
    pltpu.sync_copy(cnt_hbm, cstage)
    zeros16 = jnp.zeros((16,), jnp.float32)
    big16 = jnp.full((16,), 3e38, jnp.float32)
    zi16 = jnp.zeros((16,), jnp.int32)
    ones16 = jnp.ones((16,), jnp.int32)
    iota = lax.iota(jnp.int32, 16)

    for chunk_i in range(2):
        c = w * 2 + chunk_i

        def init_row(r, carry):
            for j in range(8):
                sl = pl.ds(j * 16, 16)
                s1[r, sl] = zeros16
                s2[r, sl] = zeros16
                mn[r, sl] = big16
                mx[r, sl] = -big16
            return carry

        lax.fori_loop(0, _CROWS, init_row, 0)

        def zfill(k, carry):
            sidx[pl.ds(k * 16, 16)] = zi16
            return carry

        lax.fori_loop(0, _LCAP // 16, zfill, 0)
        for k in range(_CROWS // 16):
            cli[pl.ds(k * 16, 16)] = zi16

        descs = []
        for t in range(_NW):
            descs.append(pltpu.async_copy(
                bkt_hbm.at[t, pl.ds(c * _CAP, _CAP)],
                lists.at[pl.ds(t * _CAP, _CAP)], sem))
        for d in descs:
            d.wait()

        def compact_t(t, cur):
            nt = jnp.minimum(_sread(cstage, t * _NCH + c), _CAP)
            ng = (nt + 15) // 16

            def g_body(gi, cur2):
                v = lists[pl.ds(t * _CAP + gi * 16, 16)]
                msk = (gi * 16 + iota) < nt
                dl = jnp.bitwise_and(v, 255)
                plsc.store_compressed(sidx.at[pl.ds(cur2, 16)],
                                      jnp.right_shift(v, 8), mask=msk)
                plsc.store_compressed(dlv.at[pl.ds(cur2, 16)], dl, mask=msk)
                plsc.addupdate_scatter(cli, [dl], ones16, mask=msk)
                return cur2 + jnp.sum(msk.astype(jnp.int32))

            return lax.fori_loop(0, ng, g_body, cur)

        m_total = lax.fori_loop(0, _NW, compact_t, 0)
        nb = (m_total + (_RB - 1)) // _RB

        def batch_body(b, carry):
            pltpu.sync_copy(c_hbm.at[sidx.at[pl.ds(b * _RB, _RB)]], rows)
            ne = jnp.minimum(m_total - b * _RB, _RB)

            def edge_body(i, carry2):
                d = _sread(dlv, b * _RB + i)
                for j in range(8):
                    sl = pl.ds(j * 16, 16)
                    g = rows[i, sl]
                    plsc.addupdate(s1.at[d, sl], g)
                    plsc.addupdate(s2.at[d, sl], g * g)
                    mn[d, sl] = jnp.minimum(mn[d, sl], g)
                    mx[d, sl] = jnp.maximum(mx[d, sl], g)
                return carry2

            lax.fori_loop(0, ne, edge_body, 0)
            return carry

        lax.fori_loop(0, nb, batch_body, 0)

        for k in range(_CROWS // 16):
            sl = pl.ds(k * 16, 16)
            clf[sl] = cli[sl].astype(jnp.float32)

        pltpu.sync_copy(s1, s1o.at[:, c, :])
        pltpu.sync_copy(s2, s2o.at[:, c, :])
        pltpu.sync_copy(mn, mno.at[:, c, :])
        pltpu.sync_copy(mx, mxo.at[:, c, :])
        pltpu.sync_copy(clf, cnto.at[c])


def kernel(x, edge_index, batch, W_pre_0, b_pre_0, W_post_0, b_post_0,
           W_lin_0, b_lin_0, W_pre_1, b_pre_1, W_post_1, b_post_1, W_lin_1,
           b_lin_1, W_mol1, b_mol1, W_mol2, b_mol2):
    src = edge_index[0]
    dst = edge_index[1]
    x_pad = jnp.pad(x, ((0, _NPAD - _N), (0, 0)))
    batch_pad = jnp.pad(batch, (0, _NPAD - _N), constant_values=_B).reshape(_NPAD, 1)
    W1_0, W2_0 = W_pre_0[:_F], W_pre_0[_F:]
    W1_1, W2_1 = W_pre_1[:_F], W_pre_1[_F:]
    r = lambda b: b.reshape(1, _F)

    A0, C0 = _t1_call(x_pad, W1_0, r(b_pre_0), W2_0)
    buckets, counts = _bucket_kernel(src, dst)

    S1a, S2a, Mna, Mxa, cnta = _accum_kernel(C0, buckets, counts)
    cnt2d = cnta.T.reshape(_NPAD, 1)
    h, A1, C1 = _combine_call(
        True, x_pad, A0, S1a.reshape(_NPAD, _F), S2a.reshape(_NPAD, _F),
        Mna.reshape(_NPAD, _F), Mxa.reshape(_NPAD, _F), cnt2d, W_post_0,
        r(b_post_0), W_lin_0, r(b_lin_0), (W1_1, r(b_pre_1), W2_1))

    S1b, S2b, Mnb, Mxb, _cntb = _accum_kernel(C1, buckets, counts)
    (h2,) = _combine_call(
        False, h, A1, S1b.reshape(_NPAD, _F), S2b.reshape(_NPAD, _F),
        Mnb.reshape(_NPAD, _F), Mxb.reshape(_NPAD, _F), cnt2d, W_post_1,
        r(b_post_1), W_lin_1, r(b_lin_1), ())

    return _pool_call(h2, batch_pad, W_mol1, r(b_mol1), W_mol2, r(b_mol2))


# speedup vs baseline: 5.3176x; 1.1459x over previous
import functools

import jax
import jax.numpy as jnp
import numpy as np
from jax import lax
from jax.experimental import pallas as pl
from jax.experimental.pallas import tpu as pltpu
from jax.experimental.pallas import tpu_sc as plsc

_N = 10000
_E = 320000
_F = 128
_B = 64

_NCH = 128
_CROWS = 80
_NPAD = _NCH * _CROWS
_NW = 32
_CPW = _NCH // _NW
_EP = _E // _NW
_NG = _EP // 16
_CAP = 256
_LCAP = 2816
_RB = 128
_SPARE = _CROWS - 1

_DEG_HIST = np.array([0]*28 + [200,600,1200,1800,2400,1800,1200,600,200] + [0]*28, dtype=np.float64)
_bins = np.arange(_DEG_HIST.shape[0], dtype=np.float64)
_AVG_DEG_LOG = float((np.log(_bins + 1.0) * _DEG_HIST).sum() / _DEG_HIST.sum())

_PREC = jax.lax.Precision.HIGHEST


def _dot(a, b):
    return jnp.dot(a, b, precision=_PREC, preferred_element_type=jnp.float32)


_RBLK = 1024
_NBLK = _NPAD // _RBLK


def _t1_body(x_ref, w1_ref, b1_ref, w2_ref, a_ref, c_ref):
    xb = x_ref[...]
    a_ref[...] = _dot(xb, w1_ref[...]) + b1_ref[...]
    c_ref[...] = _dot(xb, w2_ref[...])


def _t1_call(x_pad, W1, b1r, W2):
    full = lambda shp: pl.BlockSpec(shp, lambda i: (0, 0))
    row = pl.BlockSpec((_RBLK, _F), lambda i: (i, 0))
    return pl.pallas_call(
        _t1_body,
        grid=(_NBLK,),
        in_specs=[row, full((_F, _F)), full((1, _F)), full((_F, _F))],
        out_specs=[row, row],
        out_shape=[jax.ShapeDtypeStruct((_NPAD, _F), jnp.float32)] * 2,
    )(x_pad, W1, b1r, W2)


def _combine_body(layer0, x_ref, a_ref, s1_ref, s2_ref, mn_ref, mx_ref,
                  cnt_ref, wp_ref, bp_ref, wl_ref, bl_ref, *rest):
    if layer0:
        w1n_ref, b1n_ref, w2n_ref, h_ref, an_ref, cn_ref = rest
    else:
        (h_ref,) = rest
    xb = x_ref[...]
    A = a_ref[...]
    S1 = s1_ref[...]
    S2 = s2_ref[...]
    cnt = cnt_ref[...]
    has = cnt > 0.0
    cntc = jnp.maximum(cnt, 1.0)
    inv = 1.0 / cntc
    zero = jnp.zeros_like(A)
    mean = jnp.where(has, A + S1 * inv, zero)
    mn = jnp.where(has, A + mn_ref[...], zero)
    mx = jnp.where(has, A + mx_ref[...], zero)
    mean2 = jnp.where(has, A * A + (2.0 * A * S1 + S2) * inv, zero)
    std = jnp.sqrt(jnp.maximum(mean2 - mean * mean, 0.0) + 1e-5)
    agg = jnp.concatenate([mean, mn, mx, std], axis=-1)
    lg = jnp.log(cntc + 1.0)
    amp = lg * (1.0 / _AVG_DEG_LOG)
    att = _AVG_DEG_LOG / lg
    wp = wp_ref[...]
    out = (_dot(xb, wp[0:_F])
           + _dot(agg, wp[_F:5 * _F])
           + _dot(agg * amp, wp[5 * _F:9 * _F])
           + _dot(agg * att, wp[9 * _F:13 * _F])
           + bp_ref[...])
    out = _dot(out, wl_ref[...]) + bl_ref[...]
    if layer0:
        h = jnp.maximum(out, 0.0)
        h_ref[...] = h
        an_ref[...] = _dot(h, w1n_ref[...]) + b1n_ref[...]
        cn_ref[...] = _dot(h, w2n_ref[...])
    else:
        h_ref[...] = out


def _combine_call(layer0, x_pad, A, S1, S2, Mn, Mx, cnt2d, W_post, b_postr,
                  W_lin, b_linr, extra):
    full = lambda shp: pl.BlockSpec(shp, lambda i: (0, 0))
    row = pl.BlockSpec((_RBLK, _F), lambda i: (i, 0))
    col = pl.BlockSpec((_RBLK, 1), lambda i: (i, 0))
    in_specs = [row, row, row, row, row, row, col,
                full((13 * _F, _F)), full((1, _F)), full((_F, _F)), full((1, _F))]
    args = [x_pad, A, S1, S2, Mn, Mx, cnt2d, W_post, b_postr, W_lin, b_linr]
    if layer0:
        in_specs += [full((_F, _F)), full((1, _F)), full((_F, _F))]
        args += list(extra)
        out_specs = [row, row, row]
        out_shape = [jax.ShapeDtypeStruct((_NPAD, _F), jnp.float32)] * 3
    else:
        out_specs = [row]
        out_shape = [jax.ShapeDtypeStruct((_NPAD, _F), jnp.float32)]
    return pl.pallas_call(
        functools.partial(_combine_body, layer0),
        grid=(_NBLK,),
        in_specs=in_specs,
        out_specs=out_specs,
        out_shape=out_shape,
    )(*args)


def _pool_body(h_ref, b_ref, wm1_ref, bm1_ref, wm2_ref, bm2_ref, o_ref,
               psum, pcnt):
    i = pl.program_id(0)

    @pl.when(i == 0)
    def _():
        psum[...] = jnp.zeros_like(psum)
        pcnt[...] = jnp.zeros_like(pcnt)

    hb = h_ref[...]
    bb = b_ref[...]
    onehot = (bb == lax.broadcasted_iota(jnp.int32, (_RBLK, _B), 1)).astype(jnp.float32)
    psum[...] += lax.dot_general(onehot, hb, (((0,), (0,)), ((), ())),
                                 precision=_PREC, preferred_element_type=jnp.float32)
    pcnt[...] += jnp.broadcast_to(jnp.sum(onehot, axis=0)[:, None], (_B, _F))

    @pl.when(i == _NBLK - 1)
    def _():
        pooled = psum[...] / jnp.maximum(pcnt[...], 1.0)
        hmid = jnp.maximum(_dot(pooled, wm1_ref[...]) + bm1_ref[...], 0.0)
        o_ref[...] = _dot(hmid, wm2_ref[...]) + bm2_ref[...]


def _pool_call(h2, batch_pad, W_mol1, b_mol1r, W_mol2, b_mol2r):
    full = lambda shp: pl.BlockSpec(shp, lambda i: (0, 0))
    row = pl.BlockSpec((_RBLK, _F), lambda i: (i, 0))
    col = pl.BlockSpec((_RBLK, 1), lambda i: (i, 0))
    return pl.pallas_call(
        _pool_body,
        grid=(_NBLK,),
        in_specs=[row, col, full((_F, _F)), full((1, _F)), full((_F, _F)),
                  full((1, _F))],
        out_specs=pl.BlockSpec((_B, _F), lambda i: (0, 0)),
        out_shape=jax.ShapeDtypeStruct((_B, _F), jnp.float32),
        scratch_shapes=[pltpu.VMEM((_B, _F), jnp.float32),
                        pltpu.VMEM((_B, _F), jnp.float32)],
    )(h2, batch_pad, W_mol1, b_mol1r, W_mol2, b_mol2r)


_MESH = plsc.VectorSubcoreMesh(core_axis_name="c", subcore_axis_name="s")


def _worker_id():
    return lax.axis_index("s") * 2 + lax.axis_index("c")


def _sread(ref1d, idx):
    return plsc.load_gather(ref1d, [jnp.broadcast_to(idx, (16,))])[0]


@functools.partial(
    pl.kernel,
    out_type=[jax.ShapeDtypeStruct((_NW, _NCH * _CAP), jnp.int32),
              jax.ShapeDtypeStruct((_NW * _NCH,), jnp.int32)],
    mesh=_MESH,
    compiler_params=pltpu.CompilerParams(needs_layout_passes=False),
    scratch_types=[
        pltpu.VMEM((_EP,), jnp.int32),
        pltpu.VMEM((_EP,), jnp.int32),
        pltpu.VMEM((_NCH * _CAP,), jnp.int32),
        pltpu.VMEM((_NCH,), jnp.int32),
        pltpu.VMEM((16,), jnp.int32),
    ],
)
def _bucket_kernel(src_hbm, dst_hbm, bout, cout, sbuf, dbuf, flat, curs, kscr):
    w = _worker_id()
    base = w * _EP
    pltpu.sync_copy(src_hbm.at[pl.ds(base, _EP)], sbuf)
    pltpu.sync_copy(dst_hbm.at[pl.ds(base, _EP)], dbuf)
    zeros16 = jnp.zeros((16,), jnp.int32)
    ones16 = jnp.ones((16,), jnp.int32)
    iota = lax.iota(jnp.int32, 16)
    for i in range(_NCH // 16):
        curs[pl.ds(i * 16, 16)] = zeros16

    def body(g, carry):
        dvec = dbuf[pl.ds(g * 16, 16)]
        svec = sbuf[pl.ds(g * 16, 16)]
        cid = jnp.bitwise_and(dvec, _NCH - 1)
        dl = jnp.right_shift(dvec, 7)
        packed = svec * 128 + dl
        skey, sval = plsc.sort_key_val(cid, packed)
        kscr[pl.ds(0, 16)] = skey
        prev = plsc.load_gather(kscr, [jnp.maximum(iota - 1, 0)])
        boundary = jnp.logical_or(iota == 0, skey != prev)
        segstart = plsc.cummax(jnp.where(boundary, iota, 0))
        rank = iota - segstart
        cur = plsc.load_gather(curs, [skey])
        pos = jnp.minimum(skey * _CAP + cur + rank, skey * _CAP + (_CAP - 1))
        plsc.store_scatter(flat, [pos], sval)
        plsc.addupdate_scatter(curs, [skey], ones16)
        return carry

    lax.fori_loop(0, _NG, body, 0)
    pltpu.sync_copy(flat, bout.at[w])
    pltpu.sync_copy(curs, cout.at[pl.ds(w * _NCH, _NCH)])


@functools.partial(
    pl.kernel,
    out_type=[jax.ShapeDtypeStruct((_CROWS, _NCH, _F), jnp.float32),
              jax.ShapeDtypeStruct((_CROWS, _NCH, _F), jnp.float32),
              jax.ShapeDtypeStruct((_CROWS, _NCH, _F), jnp.float32),
              jax.ShapeDtypeStruct((_CROWS, _NCH, _F), jnp.float32),
              jax.ShapeDtypeStruct((_NCH, _CROWS), jnp.float32)],
    mesh=_MESH,
    compiler_params=pltpu.CompilerParams(needs_layout_passes=False),
    scratch_types=[
        pltpu.VMEM((_CROWS, _F), jnp.float32),
        pltpu.VMEM((_CROWS, _F), jnp.float32),
        pltpu.VMEM((_CROWS, _F), jnp.float32),
        pltpu.VMEM((_CROWS, _F), jnp.float32),
        pltpu.VMEM((_CROWS,), jnp.int32),
        pltpu.VMEM((_CROWS,), jnp.float32),
        pltpu.VMEM((_NW * _CAP,), jnp.int32),
        pltpu.VMEM((_LCAP,), jnp.int32),
        pltpu.VMEM((_LCAP,), jnp.int32),
        pltpu.VMEM((2 * _RB, _F), jnp.float32),
        pltpu.VMEM((_NW * _NCH,), jnp.int32),
        pltpu.SemaphoreType.DMA,
    ],
)
def _accum_kernel(c_hbm, bkt_hbm, cnt_hbm, s1o, s2o, mno, mxo, cnto,
                  s1, s2, mn, mx, cli, clf, lists, sidx, dlv, rows, cstage,
                  sem):
    w = _worker_id()
    pltpu.sync_copy(cnt_hbm, cstage)
    zeros16 = jnp.zeros((16,), jnp.float32)
    big16 = jnp.full((16,), 3e38, jnp.float32)
    zi16 = jnp.zeros((16,), jnp.int32)
    spare16 = jnp.full((16,), _SPARE, jnp.int32)
    ones16 = jnp.ones((16,), jnp.int32)
    iota = lax.iota(jnp.int32, 16)

    def chunk_body(chunk_i, carry0):
        c = w * _CPW + chunk_i

        def init_row(rr, carry):
            for j in range(8):
                sl = pl.ds(j * 16, 16)
                s1[rr, sl] = zeros16
                s2[rr, sl] = zeros16
                mn[rr, sl] = big16
                mx[rr, sl] = -big16
            return carry

        lax.fori_loop(0, _CROWS, init_row, 0)

        def zfill(k, carry):
            sl = pl.ds(k * 16, 16)
            sidx[sl] = zi16
            dlv[sl] = spare16
            return carry

        lax.fori_loop(0, _LCAP // 16, zfill, 0)
        for k in range(_CROWS // 16):
            cli[pl.ds(k * 16, 16)] = zi16

        descs = []
        for t in range(_NW):
            descs.append(pltpu.async_copy(
                bkt_hbm.at[t, pl.ds(c * _CAP, _CAP)],
                lists.at[pl.ds(t * _CAP, _CAP)], sem))
        for d in descs:
            d.wait()

        def compact_t(t, cur):
            nt = jnp.minimum(_sread(cstage, t * _NCH + c), _CAP)
            ng = (nt + 15) // 16

            def g_body(gi, cur2):
                v = lists[pl.ds(t * _CAP + gi * 16, 16)]
                msk = (gi * 16 + iota) < nt
                dl = jnp.bitwise_and(v, _NCH - 1)
                plsc.store_compressed(sidx.at[pl.ds(cur2, 16)],
                                      jnp.right_shift(v, 7), mask=msk)
                plsc.store_compressed(dlv.at[pl.ds(cur2, 16)], dl, mask=msk)
                plsc.addupdate_scatter(cli, [dl], ones16, mask=msk)
                return cur2 + jnp.sum(msk.astype(jnp.int32))

            return lax.fori_loop(0, ng, g_body, cur)

        m_total = lax.fori_loop(0, _NW, compact_t, 0)
        nb = (m_total + (_RB - 1)) // _RB

        def fire(b, par):
            pltpu.async_copy(
                c_hbm.at[sidx.at[pl.ds(b * _RB, _RB)]],
                rows.at[pl.ds(par * _RB, _RB), :], sem)

        @pl.when(nb > 0)
        def _():
            fire(0, 0)

        def batch_body(b, carry):
            par = jnp.bitwise_and(b, 1)

            @pl.when(b + 1 < nb)
            def _():
                fire(b + 1, 1 - par)

            pltpu.make_async_copy(
                c_hbm.at[sidx.at[pl.ds(b * _RB, _RB)]],
                rows.at[pl.ds(par * _RB, _RB), :], sem).wait()
            rbase = par * _RB

            def group_body(gi, carry2):
                ebase = gi * 16
                dvec = dlv[pl.ds(b * _RB + ebase, 16)]
                for i in range(16):
                    d = dvec[i]
                    r = rbase + ebase + i
                    for j in range(8):
                        sl = pl.ds(j * 16, 16)
                        g = rows[r, sl]
                        plsc.addupdate(s1.at[d, sl], g)
                        plsc.addupdate(s2.at[d, sl], g * g)
                        mn[d, sl] = jnp.minimum(mn[d, sl], g)
                        mx[d, sl] = jnp.maximum(mx[d, sl], g)
                return carry2

            lax.fori_loop(0, _RB // 16, group_body, 0)
            return carry

        lax.fori_loop(0, nb, batch_body, 0)

        for k in range(_CROWS // 16):
            sl = pl.ds(k * 16, 16)
            clf[sl] = cli[sl].astype(jnp.float32)

        pltpu.sync_copy(s1, s1o.at[:, c, :])
        pltpu.sync_copy(s2, s2o.at[:, c, :])
        pltpu.sync_copy(mn, mno.at[:, c, :])
        pltpu.sync_copy(mx, mxo.at[:, c, :])
        pltpu.sync_copy(clf, cnto.at[c])
        return carry0

    lax.fori_loop(0, _CPW, chunk_body, 0)


def kernel(x, edge_index, batch, W_pre_0, b_pre_0, W_post_0, b_post_0,
           W_lin_0, b_lin_0, W_pre_1, b_pre_1, W_post_1, b_post_1, W_lin_1,
           b_lin_1, W_mol1, b_mol1, W_mol2, b_mol2):
    src = edge_index[0]
    dst = edge_index[1]
    x_pad = jnp.pad(x, ((0, _NPAD - _N), (0, 0)))
    batch_pad = jnp.pad(batch, (0, _NPAD - _N), constant_values=_B).reshape(_NPAD, 1)
    W1_0, W2_0 = W_pre_0[:_F], W_pre_0[_F:]
    W1_1, W2_1 = W_pre_1[:_F], W_pre_1[_F:]
    r = lambda b: b.reshape(1, _F)

    A0, C0 = _t1_call(x_pad, W1_0, r(b_pre_0), W2_0)
    buckets, counts = _bucket_kernel(src, dst)

    S1a, S2a, Mna, Mxa, cnta = _accum_kernel(C0, buckets, counts)
    cnt2d = cnta.T.reshape(_NPAD, 1)
    h, A1, C1 = _combine_call(
        True, x_pad, A0, S1a.reshape(_NPAD, _F), S2a.reshape(_NPAD, _F),
        Mna.reshape(_NPAD, _F), Mxa.reshape(_NPAD, _F), cnt2d, W_post_0,
        r(b_post_0), W_lin_0, r(b_lin_0), (W1_1, r(b_pre_1), W2_1))

    S1b, S2b, Mnb, Mxb, _cntb = _accum_kernel(C1, buckets, counts)
    (h2,) = _combine_call(
        False, h, A1, S1b.reshape(_NPAD, _F), S2b.reshape(_NPAD, _F),
        Mnb.reshape(_NPAD, _F), Mxb.reshape(_NPAD, _F), cnt2d, W_post_1,
        r(b_post_1), W_lin_1, r(b_lin_1), ())

    return _pool_call(h2, batch_pad, W_mol1, r(b_mol1), W_mol2, r(b_mol2))
